# baseline (speedup 1.0000x reference)
import functools

import jax
import jax.numpy as jnp
from jax import lax
from jax.experimental import pallas as pl
from jax.experimental.pallas import tpu as pltpu
from jax.experimental.pallas import tpu_sc as plsc

K = 10


def _topk_body(pos_ref, lig_ref, idx_ref, *, n, m, k):
    acc = jnp.zeros((m, n), dtype=jnp.float32)
    for c in range(3):
        pc = pos_ref[0, c:c + 1, :]
        lc = lig_ref[0, :, c:c + 1]
        diff = pc - lc
        acc = acc + diff * diff
    d = jnp.sqrt(acc)

    iota = lax.broadcasted_iota(jnp.int32, (m, n), 1).astype(jnp.float32)
    big = jnp.float32(n)
    cols = []
    for _ in range(k):
        rowmin = jnp.min(d, axis=1, keepdims=True)
        cand = jnp.where(d == rowmin, iota, big)
        ik = jnp.min(cand, axis=1, keepdims=True)
        cols.append(ik)
        d = jnp.where(iota == ik, jnp.float32(jnp.inf), d)
    idx = jnp.concatenate(cols, axis=1).astype(jnp.int32)
    idx_ref[0] = idx + pl.program_id(0) * n


def _topk_call(pos_t, lig):
    b, _, n = pos_t.shape
    m = lig.shape[1]
    return pl.pallas_call(
        functools.partial(_topk_body, n=n, m=m, k=K),
        grid=(b,),
        in_specs=[
            pl.BlockSpec((1, 3, n), lambda i: (i, 0, 0)),
            pl.BlockSpec((1, m, 3), lambda i: (i, 0, 0)),
        ],
        out_specs=pl.BlockSpec((1, m, K), lambda i: (i, 0, 0)),
        out_shape=jax.ShapeDtypeStruct((b, m, K), jnp.int32),
    )(pos_t, lig)


def _sc_gather(feat_flat, gidx_flat):
    total = gidx_flat.shape[0]
    d = feat_flat.shape[1]
    info = plsc.get_sparse_core_info()
    nc, ns = info.num_cores, info.num_subcores
    nw = nc * ns
    per_w = total // nw
    ch = next(c for c in (96, 80, 64, 40, 32, 16, 8)
              if per_w % c == 0)
    n_ch = per_w // ch
    mesh = plsc.VectorSubcoreMesh(core_axis_name="c", subcore_axis_name="s")

    @functools.partial(
        pl.kernel, mesh=mesh,
        out_type=jax.ShapeDtypeStruct((total, d), jnp.float32),
        scratch_types=[
            pltpu.VMEM((ch,), jnp.int32),
            pltpu.VMEM((ch, d), jnp.float32),
            pltpu.SemaphoreType.DMA,
        ],
    )
    def gather_k(feat_hbm, idx_hbm, out_hbm, idx_v, rows_v, sem):
        wid = lax.axis_index("s") * nc + lax.axis_index("c")
        base = wid * per_w
        for c in range(n_ch):
            off = base + c * ch
            pltpu.sync_copy(idx_hbm.at[pl.ds(off, ch)], idx_v)
            pltpu.async_copy(feat_hbm.at[idx_v], rows_v, sem).wait()
            pltpu.sync_copy(rows_v, out_hbm.at[pl.ds(off, ch)])

    return gather_k(feat_flat, gidx_flat)


def _pool_body(ia_ref, ib_ref, rows_ref, x_ref):
    ii = ia_ref[0]
    jj = ib_ref[0]
    eq = (ii == jj).astype(jnp.float32)
    mult = jnp.sum(eq, axis=0, keepdims=True)
    w = 1.0 / mult
    s = jnp.sum(w)
    xv = lax.dot_general(w, rows_ref[0], (((1,), (0,)), ((), ())),
                         precision=lax.Precision.HIGHEST)
    x_ref[0] = xv / s


def _pool_call(ia, ib, rows3):
    b, p, d = rows3.shape
    return pl.pallas_call(
        _pool_body,
        grid=(b,),
        in_specs=[
            pl.BlockSpec((1, 1, p), lambda i: (i, 0, 0)),
            pl.BlockSpec((1, p, 1), lambda i: (i, 0, 0)),
            pl.BlockSpec((1, p, d), lambda i: (i, 0, 0)),
        ],
        out_specs=pl.BlockSpec((1, 1, d), lambda i: (i, 0, 0)),
        out_shape=jax.ShapeDtypeStruct((b, 1, d), jnp.float32),
    )(ia, ib, rows3)


def _mlp_body(x_ref, w1_ref, b1_ref, g_ref, be_ref, w2_ref, b2_ref, o_ref):
    h = lax.dot_general(x_ref[...], w1_ref[...], (((1,), (0,)), ((), ())),
                        precision=lax.Precision.HIGHEST) + b1_ref[...]
    mean = jnp.mean(h, axis=0, keepdims=True)
    hc = h - mean
    var = jnp.mean(hc * hc, axis=0, keepdims=True)
    hn = hc / jnp.sqrt(var + 1e-5) * g_ref[...] + be_ref[...]
    hs = hn * (1.0 / (1.0 + jnp.exp(-hn)))
    o_ref[...] = lax.dot_general(hs, w2_ref[...], (((1,), (0,)), ((), ())),
                                 precision=lax.Precision.HIGHEST) + b2_ref[...]


def _mlp_call(x, w1, b1, g, be, w2, b2):
    b, d = x.shape
    out = w2.shape[1]
    return pl.pallas_call(
        _mlp_body,
        out_shape=jax.ShapeDtypeStruct((b, out), jnp.float32),
    )(x, w1, b1, g, be, w2, b2)


def kernel(pos, features, lig_coords, W1, b1, gamma, beta, W2, b2):
    b, n, _ = pos.shape
    m = lig_coords.shape[1]
    dd = features.shape[2]

    pos_t = jnp.transpose(pos, (0, 2, 1))
    feat_flat = features.reshape(b * n, dd)
    h = b // 2
    gidx1 = _topk_call(pos_t[:h], lig_coords[:h])
    gidx2 = _topk_call(pos_t[h:], lig_coords[h:]) + h * n
    rows1 = _sc_gather(feat_flat, gidx1.reshape(h * m * K))
    rows2 = _sc_gather(feat_flat, gidx2.reshape(h * m * K))
    gflat = jnp.concatenate([gidx1.reshape(-1), gidx2.reshape(-1)])
    rows = jnp.concatenate([rows1, rows2], axis=0)

    p = m * K
    x = _pool_call(gflat.reshape(b, 1, p), gflat.reshape(b, p, 1),
                   rows.reshape(b, p, dd))
    return _mlp_call(x.reshape(b, dd), W1, b1.reshape(1, dd),
                     gamma.reshape(1, dd), beta.reshape(1, dd),
                     W2, b2.reshape(1, -1))


# speedup vs baseline: 1.0639x; 1.0639x over previous
import functools

import jax
import jax.numpy as jnp
from jax import lax
from jax.experimental import pallas as pl
from jax.experimental.pallas import tpu as pltpu
from jax.experimental.pallas import tpu_sc as plsc

K = 10


def _topk_body(pos_ref, lig_ref, idx_ref, *, n, m, k):
    acc = jnp.zeros((m, n), dtype=jnp.float32)
    for c in range(3):
        pc = pos_ref[0, c:c + 1, :]
        lc = lig_ref[0, :, c:c + 1]
        diff = pc - lc
        acc = acc + diff * diff
    d = jnp.sqrt(acc)

    iota = lax.broadcasted_iota(jnp.int32, (m, n), 1).astype(jnp.float32)
    big = jnp.float32(n)
    cols = []
    for _ in range(k):
        rowmin = jnp.min(d, axis=1, keepdims=True)
        cand = jnp.where(d == rowmin, iota, big)
        ik = jnp.min(cand, axis=1, keepdims=True)
        cols.append(ik)
        d = jnp.where(iota == ik, jnp.float32(jnp.inf), d)
    idx = jnp.concatenate(cols, axis=1).astype(jnp.int32)
    idx_ref[0] = idx + pl.program_id(0) * n


def _topk_call(pos_t, lig):
    b, _, n = pos_t.shape
    m = lig.shape[1]
    return pl.pallas_call(
        functools.partial(_topk_body, n=n, m=m, k=K),
        grid=(b,),
        in_specs=[
            pl.BlockSpec((1, 3, n), lambda i: (i, 0, 0)),
            pl.BlockSpec((1, m, 3), lambda i: (i, 0, 0)),
        ],
        out_specs=pl.BlockSpec((1, m, K), lambda i: (i, 0, 0)),
        out_shape=jax.ShapeDtypeStruct((b, m, K), jnp.int32),
    )(pos_t, lig)


def _sc_gather(feat_flat, gidx_flat):
    total = gidx_flat.shape[0]
    d = feat_flat.shape[1]
    info = plsc.get_sparse_core_info()
    nc, ns = info.num_cores, info.num_subcores
    nw = nc * ns
    per_w = total // nw
    ch = next(c for c in (96, 80, 64, 40, 32, 16, 8)
              if per_w % c == 0)
    n_ch = per_w // ch
    mesh = plsc.VectorSubcoreMesh(core_axis_name="c", subcore_axis_name="s")

    @functools.partial(
        pl.kernel, mesh=mesh,
        out_type=jax.ShapeDtypeStruct((total, d), jnp.float32),
        scratch_types=[
            pltpu.VMEM((ch,), jnp.int32),
            pltpu.VMEM((ch,), jnp.int32),
            pltpu.VMEM((ch, d), jnp.float32),
            pltpu.VMEM((ch, d), jnp.float32),
            pltpu.SemaphoreType.DMA,
            pltpu.SemaphoreType.DMA,
            pltpu.SemaphoreType.DMA,
            pltpu.SemaphoreType.DMA,
        ],
    )
    def gather_k(feat_hbm, idx_hbm, out_hbm, i0, i1, r0, r1, g0, g1, w0, w1):
        wid = lax.axis_index("s") * nc + lax.axis_index("c")
        base = wid * per_w
        idx_v = (i0, i1)
        rows_v = (r0, r1)
        gsem = (g0, g1)
        wsem = (w0, w1)
        pltpu.sync_copy(idx_hbm.at[pl.ds(base, ch)], i0)
        pltpu.async_copy(feat_hbm.at[i0], r0, g0)
        for c in range(n_ch):
            nb = (c + 1) % 2
            cb = c % 2
            if c + 1 < n_ch:
                off = base + (c + 1) * ch
                pltpu.sync_copy(idx_hbm.at[pl.ds(off, ch)], idx_v[nb])
                if c + 1 >= 2:
                    pltpu.make_async_copy(
                        rows_v[nb], out_hbm.at[pl.ds(base, ch)], wsem[nb]
                    ).wait()
                pltpu.async_copy(feat_hbm.at[idx_v[nb]], rows_v[nb], gsem[nb])
            pltpu.make_async_copy(
                feat_hbm.at[idx_v[cb]], rows_v[cb], gsem[cb]
            ).wait()
            pltpu.async_copy(
                rows_v[cb], out_hbm.at[pl.ds(base + c * ch, ch)], wsem[cb]
            )
        for c in range(max(0, n_ch - 2), n_ch):
            cb = c % 2
            pltpu.make_async_copy(
                rows_v[cb], out_hbm.at[pl.ds(base, ch)], wsem[cb]
            ).wait()

    return gather_k(feat_flat, gidx_flat)


def _pool_body(ia_ref, ib_ref, rows_ref, x_ref):
    ii = ia_ref[0]
    jj = ib_ref[0]
    eq = (ii == jj).astype(jnp.float32)
    mult = jnp.sum(eq, axis=0, keepdims=True)
    w = 1.0 / mult
    s = jnp.sum(w)
    xv = lax.dot_general(w, rows_ref[0], (((1,), (0,)), ((), ())),
                         precision=lax.Precision.HIGHEST)
    x_ref[0] = xv / s


def _pool_call(ia, ib, rows3):
    b, p, d = rows3.shape
    return pl.pallas_call(
        _pool_body,
        grid=(b,),
        in_specs=[
            pl.BlockSpec((1, 1, p), lambda i: (i, 0, 0)),
            pl.BlockSpec((1, p, 1), lambda i: (i, 0, 0)),
            pl.BlockSpec((1, p, d), lambda i: (i, 0, 0)),
        ],
        out_specs=pl.BlockSpec((1, 1, d), lambda i: (i, 0, 0)),
        out_shape=jax.ShapeDtypeStruct((b, 1, d), jnp.float32),
    )(ia, ib, rows3)


def _mlp_body(x_ref, w1_ref, b1_ref, g_ref, be_ref, w2_ref, b2_ref, o_ref):
    h = lax.dot_general(x_ref[...], w1_ref[...], (((1,), (0,)), ((), ())),
                        precision=lax.Precision.HIGHEST) + b1_ref[...]
    mean = jnp.mean(h, axis=0, keepdims=True)
    hc = h - mean
    var = jnp.mean(hc * hc, axis=0, keepdims=True)
    hn = hc / jnp.sqrt(var + 1e-5) * g_ref[...] + be_ref[...]
    hs = hn * (1.0 / (1.0 + jnp.exp(-hn)))
    o_ref[...] = lax.dot_general(hs, w2_ref[...], (((1,), (0,)), ((), ())),
                                 precision=lax.Precision.HIGHEST) + b2_ref[...]


def _mlp_call(x, w1, b1, g, be, w2, b2):
    b, d = x.shape
    out = w2.shape[1]
    return pl.pallas_call(
        _mlp_body,
        out_shape=jax.ShapeDtypeStruct((b, out), jnp.float32),
    )(x, w1, b1, g, be, w2, b2)


def kernel(pos, features, lig_coords, W1, b1, gamma, beta, W2, b2):
    b, n, _ = pos.shape
    m = lig_coords.shape[1]
    dd = features.shape[2]

    pos_t = jnp.transpose(pos, (0, 2, 1))
    gidx = _topk_call(pos_t, lig_coords)
    gflat = gidx.reshape(b * m * K)
    rows = _sc_gather(features.reshape(b * n, dd), gflat)

    p = m * K
    x = _pool_call(gflat.reshape(b, 1, p), gflat.reshape(b, p, 1),
                   rows.reshape(b, p, dd))
    return _mlp_call(x.reshape(b, dd), W1, b1.reshape(1, dd),
                     gamma.reshape(1, dd), beta.reshape(1, dd),
                     W2, b2.reshape(1, -1))


# speedup vs baseline: 1.1939x; 1.1221x over previous
import functools

import jax
import jax.numpy as jnp
from jax import lax
from jax.experimental import pallas as pl
from jax.experimental.pallas import tpu as pltpu
from jax.experimental.pallas import tpu_sc as plsc

K = 10


def _topk_body(pos_ref, lig_ref, idx_ref, *, n, m, k):
    acc = jnp.zeros((m, n), dtype=jnp.float32)
    for c in range(3):
        pc = pos_ref[0, c:c + 1, :]
        lc = lig_ref[0, :, c:c + 1]
        diff = pc - lc
        acc = acc + diff * diff
    d = acc

    iota = lax.broadcasted_iota(jnp.int32, (m, n), 1).astype(jnp.float32)
    big = jnp.float32(n)
    cols = []
    ik = None
    for _ in range(k):
        if ik is not None:
            d = jnp.where(iota == ik, jnp.float32(jnp.inf), d)
        rowmin = jnp.min(d, axis=1, keepdims=True)
        cand = jnp.where(d == rowmin, iota, big)
        ik = jnp.min(cand, axis=1, keepdims=True)
        cols.append(ik)
    idx = jnp.concatenate(cols, axis=1).astype(jnp.int32)
    idx_ref[0] = idx + pl.program_id(0) * n


def _topk_call(pos_t, lig):
    b, _, n = pos_t.shape
    m = lig.shape[1]
    return pl.pallas_call(
        functools.partial(_topk_body, n=n, m=m, k=K),
        grid=(b,),
        in_specs=[
            pl.BlockSpec((1, 3, n), lambda i: (i, 0, 0)),
            pl.BlockSpec((1, m, 3), lambda i: (i, 0, 0)),
        ],
        out_specs=pl.BlockSpec((1, m, K), lambda i: (i, 0, 0)),
        out_shape=jax.ShapeDtypeStruct((b, m, K), jnp.int32),
    )(pos_t, lig)


def _sc_gather(feat_flat, gidx_flat):
    total = gidx_flat.shape[0]
    d = feat_flat.shape[1]
    info = plsc.get_sparse_core_info()
    nc, ns = info.num_cores, info.num_subcores
    nw = nc * ns
    per_w = total // nw
    ch = next(c for c in (96, 80, 64, 40, 32, 16, 8)
              if per_w % c == 0)
    n_ch = per_w // ch
    mesh = plsc.VectorSubcoreMesh(core_axis_name="c", subcore_axis_name="s")

    @functools.partial(
        pl.kernel, mesh=mesh,
        out_type=jax.ShapeDtypeStruct((total, d), jnp.float32),
        scratch_types=[
            pltpu.VMEM((ch,), jnp.int32),
            pltpu.VMEM((ch,), jnp.int32),
            pltpu.VMEM((ch, d), jnp.float32),
            pltpu.VMEM((ch, d), jnp.float32),
            pltpu.SemaphoreType.DMA,
            pltpu.SemaphoreType.DMA,
            pltpu.SemaphoreType.DMA,
            pltpu.SemaphoreType.DMA,
        ],
    )
    def gather_k(feat_hbm, idx_hbm, out_hbm, i0, i1, r0, r1, g0, g1, w0, w1):
        wid = lax.axis_index("s") * nc + lax.axis_index("c")
        base = wid * per_w
        idx_v = (i0, i1)
        rows_v = (r0, r1)
        gsem = (g0, g1)
        wsem = (w0, w1)
        pltpu.sync_copy(idx_hbm.at[pl.ds(base, ch)], i0)
        pltpu.async_copy(feat_hbm.at[i0], r0, g0)
        for c in range(n_ch):
            nb = (c + 1) % 2
            cb = c % 2
            if c + 1 < n_ch:
                off = base + (c + 1) * ch
                pltpu.sync_copy(idx_hbm.at[pl.ds(off, ch)], idx_v[nb])
                if c + 1 >= 2:
                    pltpu.make_async_copy(
                        rows_v[nb], out_hbm.at[pl.ds(base, ch)], wsem[nb]
                    ).wait()
                pltpu.async_copy(feat_hbm.at[idx_v[nb]], rows_v[nb], gsem[nb])
            pltpu.make_async_copy(
                feat_hbm.at[idx_v[cb]], rows_v[cb], gsem[cb]
            ).wait()
            pltpu.async_copy(
                rows_v[cb], out_hbm.at[pl.ds(base + c * ch, ch)], wsem[cb]
            )
        for c in range(max(0, n_ch - 2), n_ch):
            cb = c % 2
            pltpu.make_async_copy(
                rows_v[cb], out_hbm.at[pl.ds(base, ch)], wsem[cb]
            ).wait()

    return gather_k(feat_flat, gidx_flat)


def _pool_body(ia_ref, ib_ref, rows_ref, x_ref):
    ii = ia_ref[0]
    jj = ib_ref[0]
    eq = (ii == jj).astype(jnp.float32)
    mult = jnp.sum(eq, axis=0, keepdims=True)
    w = 1.0 / mult
    s = jnp.sum(w)
    xv = lax.dot_general(w, rows_ref[0], (((1,), (0,)), ((), ())),
                         precision=lax.Precision.HIGHEST)
    x_ref[0] = xv / s


def _pool_call(ia, ib, rows3):
    b, p, d = rows3.shape
    return pl.pallas_call(
        _pool_body,
        grid=(b,),
        in_specs=[
            pl.BlockSpec((1, 1, p), lambda i: (i, 0, 0)),
            pl.BlockSpec((1, p, 1), lambda i: (i, 0, 0)),
            pl.BlockSpec((1, p, d), lambda i: (i, 0, 0)),
        ],
        out_specs=pl.BlockSpec((1, 1, d), lambda i: (i, 0, 0)),
        out_shape=jax.ShapeDtypeStruct((b, 1, d), jnp.float32),
    )(ia, ib, rows3)


def _mlp_body(x_ref, w1_ref, b1_ref, g_ref, be_ref, w2_ref, b2_ref, o_ref):
    h = lax.dot_general(x_ref[...], w1_ref[...], (((1,), (0,)), ((), ())),
                        precision=lax.Precision.HIGHEST) + b1_ref[...]
    mean = jnp.mean(h, axis=0, keepdims=True)
    hc = h - mean
    var = jnp.mean(hc * hc, axis=0, keepdims=True)
    hn = hc / jnp.sqrt(var + 1e-5) * g_ref[...] + be_ref[...]
    hs = hn * (1.0 / (1.0 + jnp.exp(-hn)))
    o_ref[...] = lax.dot_general(hs, w2_ref[...], (((1,), (0,)), ((), ())),
                                 precision=lax.Precision.HIGHEST) + b2_ref[...]


def _mlp_call(x, w1, b1, g, be, w2, b2):
    b, d = x.shape
    out = w2.shape[1]
    return pl.pallas_call(
        _mlp_body,
        out_shape=jax.ShapeDtypeStruct((b, out), jnp.float32),
    )(x, w1, b1, g, be, w2, b2)


def kernel(pos, features, lig_coords, W1, b1, gamma, beta, W2, b2):
    b, n, _ = pos.shape
    m = lig_coords.shape[1]
    dd = features.shape[2]

    pos_t = jnp.transpose(pos, (0, 2, 1))
    gidx = _topk_call(pos_t, lig_coords)
    gflat = gidx.reshape(b * m * K)
    rows = _sc_gather(features.reshape(b * n, dd), gflat)

    p = m * K
    x = _pool_call(gflat.reshape(b, 1, p), gflat.reshape(b, p, 1),
                   rows.reshape(b, p, dd))
    return _mlp_call(x.reshape(b, dd), W1, b1.reshape(1, dd),
                     gamma.reshape(1, dd), beta.reshape(1, dd),
                     W2, b2.reshape(1, -1))
